# baseline (speedup 1.0000x reference)
import functools

import jax
import jax.numpy as jnp
from jax import lax
from jax.experimental import pallas as pl
from jax.experimental.pallas import tpu as pltpu
from jax.experimental.pallas import tpu_sc as plsc

NC = 2
NS = 16
NW = NC * NS
CHUNK = 128
TCOLS = 1024


def _tc_transpose(t_T):
    d, v = t_T.shape
    steps = -(-v // TCOLS)
    h = TCOLS // 2

    def body(x_ref, o_ref):
        x = x_ref[...]
        o_ref[...] = jnp.concatenate([x[:, :h].T, x[:, h:].T], axis=1)

    return pl.pallas_call(
        body,
        grid=(steps,),
        in_specs=[pl.BlockSpec((d, TCOLS), lambda j: (0, j))],
        out_specs=pl.BlockSpec((TCOLS // 2, 2 * d), lambda j: (j, 0)),
        out_shape=jax.ShapeDtypeStruct((steps * h, 2 * d), jnp.float32),
    )(t_T)


def _sc_gather(table_lin, idx2, n, d):
    chunks_per_w = (n // CHUNK) // NW
    mesh = plsc.VectorSubcoreMesh(core_axis_name="c", subcore_axis_name="s")

    @functools.partial(
        pl.kernel,
        mesh=mesh,
        compiler_params=pltpu.CompilerParams(use_tc_tiling_on_sc=False),
        out_type=jax.ShapeDtypeStruct((n, d), jnp.float32),
        scratch_types=[
            pltpu.VMEM((chunks_per_w, CHUNK), jnp.int32),
            pltpu.VMEM((CHUNK, d), jnp.float32),
            pltpu.VMEM((CHUNK, d), jnp.float32),
            pltpu.SemaphoreType.DMA,
            pltpu.SemaphoreType.DMA,
        ],
    )
    def k(tab_hbm, idx_hbm, out_hbm, idx_v, rows_a, rows_b, sem_a, sem_b):
        wid = lax.axis_index("s") * NC + lax.axis_index("c")
        pltpu.sync_copy(idx_hbm.at[pl.ds(wid * chunks_per_w, chunks_per_w)], idx_v)

        @pl.loop(0, chunks_per_w, step=2)
        def _(j):
            pltpu.async_copy(tab_hbm.at[idx_v.at[j]], rows_a, sem_a).wait()
            cp_a = pltpu.make_async_copy(
                rows_a, out_hbm.at[pl.ds((wid * chunks_per_w + j) * CHUNK, CHUNK)], sem_a)
            cp_a.start()
            pltpu.async_copy(tab_hbm.at[idx_v.at[j + 1]], rows_b, sem_b).wait()
            cp_b = pltpu.make_async_copy(
                rows_b, out_hbm.at[pl.ds((wid * chunks_per_w + j + 1) * CHUNK, CHUNK)], sem_b)
            cp_b.start()
            cp_a.wait()
            cp_b.wait()

    return k(table_lin, idx2)


def _tc_loss(es2, v2, b, n_neg, d, blk):
    steps = (b // 2) // blk

    def body(s_ref, v_ref, o_ref):
        i = pl.program_id(0)
        es = s_ref[...]
        total = jnp.float32(0.0)
        for n in range(1 + n_neg):
            prod = es * v_ref[n]
            s_lo = jnp.sum(prod[:, :d], axis=1)
            s_hi = jnp.sum(prod[:, d:], axis=1)
            if n == 0:
                total += jnp.sum(jax.nn.log_sigmoid(s_lo))
                total += jnp.sum(jax.nn.log_sigmoid(s_hi))
            else:
                total += jnp.sum(jax.nn.log_sigmoid(-s_lo))
                total += jnp.sum(jax.nn.log_sigmoid(-s_hi))

        @pl.when(i == 0)
        def _():
            o_ref[0, 0] = 0.0

        o_ref[0, 0] += -total

    return pl.pallas_call(
        body,
        grid=(steps,),
        in_specs=[
            pl.BlockSpec((blk, 2 * d), lambda i: (i, 0)),
            pl.BlockSpec((1 + n_neg, blk, 2 * d), lambda i: (0, i, 0)),
        ],
        out_specs=pl.BlockSpec((1, 1), lambda i: (0, 0),
                               memory_space=pltpu.MemorySpace.SMEM),
        out_shape=jax.ShapeDtypeStruct((1, 1), jnp.float32),
    )(es2, v2)


def kernel(pos_u, pos_v, neg_v, rightsense, v_emb, sense_emb):
    b = pos_u.shape[0]
    n_neg = neg_v.shape[1]
    d = v_emb.shape[1]
    k_senses = sense_emb.shape[0] // v_emb.shape[0]

    rs = jnp.asarray(rightsense, dtype=jnp.int32)
    sense_idx = pos_u.astype(jnp.int32) * jnp.int32(k_senses) + rs
    v_idx = jnp.concatenate([pos_v[None, :], neg_v.T], axis=0).reshape(-1)
    n_v = (1 + n_neg) * b

    def _permute_idx(idx):
        i = idx % TCOLS
        return idx - i + 2 * (i % (TCOLS // 2)) + i // (TCOLS // 2)

    sense_idx = _permute_idx(sense_idx)
    v_idx = _permute_idx(v_idx)

    sidx2 = sense_idx.reshape(b // CHUNK, CHUNK)
    vidx2 = v_idx.reshape(n_v // CHUNK, CHUNK)

    v_lin2 = _tc_transpose(v_emb.T)
    v_lin = v_lin2.reshape(-1, d)
    v_rows = _sc_gather(v_lin, vidx2, n_v, d)

    s_lin2 = _tc_transpose(sense_emb.T)
    s_lin = s_lin2.reshape(-1, d)
    sense_rows = _sc_gather(s_lin, sidx2, b, d)

    es2 = sense_rows.reshape(b // 2, 2 * d)
    v2 = v_rows.reshape(1 + n_neg, b // 2, 2 * d)
    out = _tc_loss(es2, v2, b, n_neg, d, blk=1024)
    return out.reshape(())


# speedup vs baseline: 1.5581x; 1.5581x over previous
import functools

import jax
import jax.numpy as jnp
from jax import lax
from jax.experimental import pallas as pl
from jax.experimental.pallas import tpu as pltpu
from jax.experimental.pallas import tpu_sc as plsc

NC = 2
NS = 16
NW = NC * NS
CHUNK = 128
TCOLS = 2048


def _tc_transpose(t_T):
    d, v = t_T.shape
    steps = -(-v // TCOLS)
    h = TCOLS // 2

    def body(x_ref, o_ref):
        x = x_ref[...]
        parts = []
        for j in range(TCOLS // 256):
            sq = jnp.concatenate(
                [x[:, j * 256:j * 256 + 128], x[:, j * 256 + 128:(j + 1) * 256]],
                axis=0)
            parts.append(sq.T)
        o_ref[...] = jnp.concatenate(parts, axis=0)

    return pl.pallas_call(
        body,
        grid=(steps,),
        in_specs=[pl.BlockSpec((d, TCOLS), lambda j: (0, j))],
        out_specs=pl.BlockSpec((TCOLS // 2, 2 * d), lambda j: (j, 0)),
        out_shape=jax.ShapeDtypeStruct((steps * h, 2 * d), jnp.float32),
    )(t_T)


def _sc_gather(table_lin, idx2, n, d):
    chunks_per_w = (n // CHUNK) // NW
    mesh = plsc.VectorSubcoreMesh(core_axis_name="c", subcore_axis_name="s")

    @functools.partial(
        pl.kernel,
        mesh=mesh,
        compiler_params=pltpu.CompilerParams(use_tc_tiling_on_sc=False),
        out_type=jax.ShapeDtypeStruct((n, d), jnp.float32),
        scratch_types=[
            pltpu.VMEM((chunks_per_w, CHUNK), jnp.int32),
            pltpu.VMEM((CHUNK, d), jnp.float32),
            pltpu.VMEM((CHUNK, d), jnp.float32),
            pltpu.SemaphoreType.DMA,
            pltpu.SemaphoreType.DMA,
        ],
    )
    def k(tab_hbm, idx_hbm, out_hbm, idx_v, rows_a, rows_b, sem_a, sem_b):
        wid = lax.axis_index("s") * NC + lax.axis_index("c")
        pltpu.sync_copy(idx_hbm.at[pl.ds(wid * chunks_per_w, chunks_per_w)], idx_v)

        @pl.loop(0, chunks_per_w, step=2)
        def _(j):
            pltpu.async_copy(tab_hbm.at[idx_v.at[j]], rows_a, sem_a).wait()
            cp_a = pltpu.make_async_copy(
                rows_a, out_hbm.at[pl.ds((wid * chunks_per_w + j) * CHUNK, CHUNK)], sem_a)
            cp_a.start()
            pltpu.async_copy(tab_hbm.at[idx_v.at[j + 1]], rows_b, sem_b).wait()
            cp_b = pltpu.make_async_copy(
                rows_b, out_hbm.at[pl.ds((wid * chunks_per_w + j + 1) * CHUNK, CHUNK)], sem_b)
            cp_b.start()
            cp_a.wait()
            cp_b.wait()

    return k(table_lin, idx2)


def _tc_loss(es2, v2, b, n_neg, d, blk):
    steps = (b // 2) // blk

    def body(s_ref, v_ref, o_ref):
        i = pl.program_id(0)
        es = s_ref[...]
        total = jnp.float32(0.0)
        for n in range(1 + n_neg):
            prod = es * v_ref[n]
            s_lo = jnp.sum(prod[:, :d], axis=1)
            s_hi = jnp.sum(prod[:, d:], axis=1)
            if n == 0:
                total += jnp.sum(jax.nn.log_sigmoid(s_lo))
                total += jnp.sum(jax.nn.log_sigmoid(s_hi))
            else:
                total += jnp.sum(jax.nn.log_sigmoid(-s_lo))
                total += jnp.sum(jax.nn.log_sigmoid(-s_hi))

        @pl.when(i == 0)
        def _():
            o_ref[0, 0] = 0.0

        o_ref[0, 0] += -total

    return pl.pallas_call(
        body,
        grid=(steps,),
        in_specs=[
            pl.BlockSpec((blk, 2 * d), lambda i: (i, 0)),
            pl.BlockSpec((1 + n_neg, blk, 2 * d), lambda i: (0, i, 0)),
        ],
        out_specs=pl.BlockSpec((1, 1), lambda i: (0, 0),
                               memory_space=pltpu.MemorySpace.SMEM),
        out_shape=jax.ShapeDtypeStruct((1, 1), jnp.float32),
    )(es2, v2)


def kernel(pos_u, pos_v, neg_v, rightsense, v_emb, sense_emb):
    b = pos_u.shape[0]
    n_neg = neg_v.shape[1]
    d = v_emb.shape[1]
    k_senses = sense_emb.shape[0] // v_emb.shape[0]

    rs = jnp.asarray(rightsense, dtype=jnp.int32)
    sense_idx = pos_u.astype(jnp.int32) * jnp.int32(k_senses) + rs
    v_idx = jnp.concatenate([pos_v[None, :], neg_v.T], axis=0).reshape(-1)
    n_v = (1 + n_neg) * b

    def _permute_idx(idx):
        c = idx % 256
        return idx - c + 2 * (c % 128) + c // 128

    sense_idx = _permute_idx(sense_idx)
    v_idx = _permute_idx(v_idx)

    sidx2 = sense_idx.reshape(b // CHUNK, CHUNK)
    vidx2 = v_idx.reshape(n_v // CHUNK, CHUNK)

    v_lin2 = _tc_transpose(v_emb.T)
    v_lin = v_lin2.reshape(-1, d)
    v_rows = _sc_gather(v_lin, vidx2, n_v, d)

    s_t, _ = lax.optimization_barrier((sense_emb.T, v_lin))
    s_lin2 = _tc_transpose(s_t)
    s_lin = s_lin2.reshape(-1, d)
    sense_rows = _sc_gather(s_lin, sidx2, b, d)

    es2 = sense_rows.reshape(b // 2, 2 * d)
    v2 = v_rows.reshape(1 + n_neg, b // 2, 2 * d)
    out = _tc_loss(es2, v2, b, n_neg, d, blk=1024)
    return out.reshape(())
